# baseline (speedup 1.0000x reference)
import functools

import jax
import jax.numpy as jnp
from jax import lax
from jax.experimental import pallas as pl
from jax.experimental.pallas import tpu as pltpu
from jax.experimental.pallas import tpu_sc as plsc

NC = 2
NS = 16
NW = NC * NS
L = 16
D = 128
DV = D // L
K = 32
KB = 80
NEG_SLOPE = 0.01


def _leaky(v):
    return jnp.maximum(v, v * NEG_SLOPE)


def _hsum_bcast(v):
    total = plsc.cumsum(v)[L - 1]
    return jnp.full((L,), total, jnp.float32)


def _edge_body(x_hbm, sd_hbm, wf_hbm, wb_hbm, part_hbm, zpart_hbm,
               acc, zacc, idx2_0, idx2_1, idx2_2, idx2_3,
               xs0, xs1, xd0, xd1, rows0, rows1,
               wbuf_f, wbuf_r, zloc, zzero, idxrow, wfb, wbb,
               sem_i0, sem_i1, sem_i2, sem_i3,
               sem_s0, sem_s1, sem_d0, sem_d1, sem_c0, sem_c1,
               np_pad, e_pad, ch):
    c = lax.axis_index("c")
    s = lax.axis_index("s")
    wid = c * NS + s
    rows_per_tile = np_pad // NS
    zrows = np_pad // D
    zrows_per_tile = 8
    zero = jnp.zeros((L,), jnp.float32)
    lanes = lax.iota(jnp.int32, L)
    idx2 = [idx2_0, idx2_1, idx2_2, idx2_3]
    xs = [xs0, xs1]
    xd = [xd0, xd1]
    rows = [rows0, rows1]
    sem_i = [sem_i0, sem_i1, sem_i2, sem_i3]
    sem_s = [sem_s0, sem_s1]
    sem_d = [sem_d0, sem_d1]
    sem_c = [sem_c0, sem_c1]

    def zero_row(r, carry):
        for j in range(DV):
            rows0[r, pl.ds(j * L, L)] = zero
        return carry

    lax.fori_loop(0, 2 * K, zero_row, 0)

    def zero_zz(r, carry):
        for j in range(DV):
            zzero[r, pl.ds(j * L, L)] = zero
        return carry

    lax.fori_loop(0, zrows_per_tile, zero_zz, 0)

    tile_base = s * rows_per_tile
    for b in range(rows_per_tile // (2 * K)):
        pltpu.sync_copy(rows0, acc.at[pl.ds(tile_base + b * 2 * K, 2 * K)])

    @pl.when(s < zrows // zrows_per_tile)
    def _():
        pltpu.sync_copy(zzero,
                        zacc.at[pl.ds(s * zrows_per_tile, zrows_per_tile)])

    def zero_z(r, carry):
        for j in range(DV):
            zloc[r, pl.ds(j * L, L)] = zero
        return carry

    lax.fori_loop(0, zrows, zero_z, 0)
    plsc.subcore_barrier()

    pltpu.sync_copy(wf_hbm, wfb)
    pltpu.sync_copy(wb_hbm, wbb)
    wfv = [wfb[pl.ds(j * L, L)] for j in range(DV)]
    wbv = [wbb[pl.ds(j * L, L)] for j in range(DV)]

    epw = ch * K
    base = wid * epw

    def issue_idx(ci, bi):
        off = base + ci * K
        pltpu.async_copy(sd_hbm.at[pl.ds(e_pad + off, K)],
                         idx2[bi].at[pl.ds(0, K)], sem_i[bi])
        pltpu.async_copy(sd_hbm.at[pl.ds(off, K)],
                         idx2[bi].at[pl.ds(K, K)], sem_i[bi])

    def wait_idx(bi):
        pltpu.make_async_copy(sd_hbm.at[pl.ds(0, K)],
                              idx2[bi].at[pl.ds(0, K)], sem_i[bi]).wait()
        pltpu.make_async_copy(sd_hbm.at[pl.ds(0, K)],
                              idx2[bi].at[pl.ds(K, K)], sem_i[bi]).wait()

    def start_gathers(b, bi):
        pltpu.async_copy(x_hbm.at[idx2[bi].at[pl.ds(0, K)]], xd[b], sem_d[b])
        pltpu.async_copy(x_hbm.at[idx2[bi].at[pl.ds(K, K)]], xs[b], sem_s[b])

    def wait_gathers(b, bi):
        pltpu.make_async_copy(x_hbm.at[idx2[bi].at[pl.ds(0, K)]], xd[b],
                              sem_d[b]).wait()
        pltpu.make_async_copy(x_hbm.at[idx2[bi].at[pl.ds(K, K)]], xs[b],
                              sem_s[b]).wait()

    def wait_scatter(b, bi):
        pltpu.make_async_copy(rows[b], acc.at[idx2[bi]], sem_c[b]).wait()

    issue_idx(0, 0)
    wait_idx(0)
    start_gathers(0, 0)
    issue_idx(1, 1)

    def process(ci, b, bi):
        nb = 1 - b
        nbi = (bi + 1) % 4

        @pl.when(ci + 1 < ch)
        def _():
            wait_idx(nbi)
            start_gathers(nb, nbi)

        wait_gathers(b, bi)

        @plsc.parallel_loop(0, K, 1, unroll=2)
        def edge_body(e):
            accf = xs[b][e, pl.ds(0, L)] * xd[b][e, pl.ds(0, L)] * wfv[0]
            accb = xs[b][e, pl.ds(0, L)] * xd[b][e, pl.ds(0, L)] * wbv[0]
            for j in range(1, DV):
                p = xs[b][e, pl.ds(j * L, L)] * xd[b][e, pl.ds(j * L, L)]
                accf = accf + p * wfv[j]
                accb = accb + p * wbv[j]
            wfe = jnp.exp(_leaky(_hsum_bcast(accf)))
            wbe = jnp.exp(_leaky(_hsum_bcast(accb)))
            for j in range(DV):
                rows[b][e, pl.ds(j * L, L)] = xs[b][e, pl.ds(j * L, L)] * wfe
                rows[b][K + e, pl.ds(j * L, L)] = (xd[b][e, pl.ds(j * L, L)]
                                                   * wbe)
            wbuf_f[pl.ds(e * L, L)] = wfe
            wbuf_r[pl.ds(e * L, L)] = wbe

        for g in range(K // L):
            ivd = idx2[bi][pl.ds(g * L, L)]
            ivs = idx2[bi][pl.ds(K + g * L, L)]
            iw = lanes * L + (g * L * L)
            wvf = plsc.load_gather(wbuf_f, [iw])
            wvr = plsc.load_gather(wbuf_r, [iw])
            plsc.addupdate_scatter(zloc, [ivd >> 7, ivd & 127], wvf)
            plsc.addupdate_scatter(zloc, [ivs >> 7, ivs & 127], wvr)

        @pl.when(ci >= 1)
        def _():
            wait_scatter(1 - b, (bi + 3) % 4)

        pltpu.async_copy(rows[b], acc.at[idx2[bi]], sem_c[b], add=True)

        @pl.when(ci + 2 < ch)
        def _():
            issue_idx(ci + 2, (bi + 2) % 4)

    def quad_body(t, carry):
        for q in range(4):
            process(4 * t + q, q % 2, q)
        return carry

    nquad = ch // 4
    lax.fori_loop(0, nquad, quad_body, 0)
    for r in range(ch % 4):
        process(nquad * 4 + r, r % 2, r)
    wait_scatter((ch - 1) % 2, (ch - 1) % 4)

    for gg in range(zrows // L):
        idxrow[pl.ds(gg * L, L)] = lanes + gg * L
    pltpu.sync_copy(zloc, zacc.at[idxrow], add=True)

    plsc.subcore_barrier()

    pltpu.sync_copy(acc.at[pl.ds(tile_base, rows_per_tile)],
                    part_hbm.at[pl.ds(c * np_pad + tile_base, rows_per_tile)])

    @pl.when(s < zrows // zrows_per_tile)
    def _():
        pltpu.sync_copy(
            zacc.at[pl.ds(s * zrows_per_tile, zrows_per_tile)],
            zpart_hbm.at[pl.ds(c * zrows + s * zrows_per_tile,
                               zrows_per_tile)])


def _combine_body(part_hbm, zpart_hbm, x_hbm, wf_hbm, mask_hbm, out_hbm,
                  a0, a1, xb, mb, ob, zb0, zb1, zt, wfb, np_pad):
    c = lax.axis_index("c")
    s = lax.axis_index("s")
    wid = c * NS + s
    rows_per_w = np_pad // NW
    base = wid * rows_per_w

    pltpu.sync_copy(wf_hbm, wfb)
    wfv = [wfb[pl.ds(j * L, L)] for j in range(DV)]

    def chunk_body(cb, carry):
        off = base + cb * KB
        pltpu.sync_copy(part_hbm.at[pl.ds(off, KB)], a0)
        pltpu.sync_copy(part_hbm.at[pl.ds(np_pad + off, KB)], a1)
        pltpu.sync_copy(x_hbm.at[pl.ds(off, KB)], xb)
        pltpu.sync_copy(mask_hbm.at[pl.ds(off, KB)], mb.at[pl.ds(0, KB)])
        pltpu.sync_copy(zpart_hbm.at[pl.ds(off, KB)], zb0)
        pltpu.sync_copy(zpart_hbm.at[pl.ds(np_pad + off, KB)], zb1)

        for g in range(KB // L):
            zt[pl.ds(g * L, L)] = (zb0[pl.ds(g * L, L)]
                                   + zb1[pl.ds(g * L, L)])

        def row_body(r, carry2):
            xv = [xb[r, pl.ds(j * L, L)] for j in range(DV)]
            accw = xv[0] * xv[0] * wfv[0]
            for j in range(1, DV):
                accw = accw + xv[j] * xv[j] * wfv[j]
            wsv = jnp.exp(_leaky(_hsum_bcast(accw)))
            z0 = zt[pl.ds(r, L)][0]
            zv = jnp.full((L,), z0, jnp.float32) + wsv
            m0 = mb[pl.ds(r, L)][0]
            pred = jnp.full((L,), m0, jnp.int32) == 1
            for j in range(DV):
                num = a0[r, pl.ds(j * L, L)] + a1[r, pl.ds(j * L, L)] + wsv * xv[j]
                ob[r, pl.ds(j * L, L)] = jnp.where(pred, num / zv + xv[j],
                                                   xv[j] + xv[j])
            return carry2

        lax.fori_loop(0, KB, row_body, 0)
        pltpu.sync_copy(ob, out_hbm.at[pl.ds(off, KB)])
        return carry

    lax.fori_loop(0, rows_per_w // KB, chunk_body, 0)


def kernel(x, edge_index, batch, local_sess_avg, mt_sess_masks,
           W_forward, W_backward, W_alpha):
    n, d = x.shape
    assert d == D
    e = edge_index.shape[1]
    step = NW * KB
    np_pad = ((n + step - 1) // step) * step
    ch = (e + NW * K - 1) // (NW * K)
    e_pad = ch * K * NW
    zrows = np_pad // D

    src = edge_index[0].astype(jnp.int32)
    dst = edge_index[1].astype(jnp.int32)
    pad_idx = jnp.full((e_pad - e,), n, jnp.int32)
    sd = jnp.concatenate([src, pad_idx, dst, pad_idx], axis=0)
    x_pad = jnp.pad(x, ((0, np_pad - n), (0, 0)))
    mask_pad = jnp.pad(mt_sess_masks.astype(jnp.int32), (0, np_pad - n))
    wf = W_forward[:, 0]
    wb = W_backward[:, 0]

    mesh = plsc.VectorSubcoreMesh(core_axis_name="c", subcore_axis_name="s")

    edge_call = pl.kernel(
        functools.partial(_edge_body, np_pad=np_pad, e_pad=e_pad, ch=ch),
        out_type=(jax.ShapeDtypeStruct((NC * np_pad, D), jnp.float32),
                  jax.ShapeDtypeStruct((NC * zrows, D), jnp.float32)),
        mesh=mesh,
        compiler_params=pltpu.CompilerParams(needs_layout_passes=False),
        scratch_types=[
            pltpu.VMEM_SHARED((np_pad, D), jnp.float32),
            pltpu.VMEM_SHARED((zrows, D), jnp.float32),
            pltpu.VMEM((2 * K,), jnp.int32),
            pltpu.VMEM((2 * K,), jnp.int32),
            pltpu.VMEM((2 * K,), jnp.int32),
            pltpu.VMEM((2 * K,), jnp.int32),
            pltpu.VMEM((K, D), jnp.float32),
            pltpu.VMEM((K, D), jnp.float32),
            pltpu.VMEM((K, D), jnp.float32),
            pltpu.VMEM((K, D), jnp.float32),
            pltpu.VMEM((2 * K, D), jnp.float32),
            pltpu.VMEM((2 * K, D), jnp.float32),
            pltpu.VMEM((K * L,), jnp.float32),
            pltpu.VMEM((K * L,), jnp.float32),
            pltpu.VMEM((zrows, D), jnp.float32),
            pltpu.VMEM((8, D), jnp.float32),
            pltpu.VMEM((zrows,), jnp.int32),
            pltpu.VMEM((D,), jnp.float32),
            pltpu.VMEM((D,), jnp.float32),
            pltpu.SemaphoreType.DMA,
            pltpu.SemaphoreType.DMA,
            pltpu.SemaphoreType.DMA,
            pltpu.SemaphoreType.DMA,
            pltpu.SemaphoreType.DMA,
            pltpu.SemaphoreType.DMA,
            pltpu.SemaphoreType.DMA,
            pltpu.SemaphoreType.DMA,
            pltpu.SemaphoreType.DMA,
            pltpu.SemaphoreType.DMA,
        ],
    )
    part, zpart = edge_call(x_pad, sd, wf, wb)
    zflat = zpart.reshape(NC * np_pad)

    combine_call = pl.kernel(
        functools.partial(_combine_body, np_pad=np_pad),
        out_type=jax.ShapeDtypeStruct((np_pad, D), jnp.float32),
        mesh=mesh,
        compiler_params=pltpu.CompilerParams(needs_layout_passes=False),
        scratch_types=[
            pltpu.VMEM((KB, D), jnp.float32),
            pltpu.VMEM((KB, D), jnp.float32),
            pltpu.VMEM((KB, D), jnp.float32),
            pltpu.VMEM((KB + L,), jnp.int32),
            pltpu.VMEM((KB, D), jnp.float32),
            pltpu.VMEM((KB,), jnp.float32),
            pltpu.VMEM((KB,), jnp.float32),
            pltpu.VMEM((KB + L,), jnp.float32),
            pltpu.VMEM((D,), jnp.float32),
        ],
    )
    out_pad = combine_call(part, zflat, x_pad, wf, mask_pad)
    return out_pad[:n]


# speedup vs baseline: 1.5788x; 1.5788x over previous
import functools

import jax
import jax.numpy as jnp
from jax import lax
from jax.experimental import pallas as pl
from jax.experimental.pallas import tpu as pltpu
from jax.experimental.pallas import tpu_sc as plsc

NC = 2
NS = 16
NW = NC * NS
L = 16
D = 128
DV = D // L
K = 32
KB = 80
NEG_SLOPE = 0.01


def _leaky(v):
    return jnp.maximum(v, v * NEG_SLOPE)


def _hsum_bcast(v):
    total = plsc.cumsum(v)[L - 1]
    return jnp.full((L,), total, jnp.float32)


def _edge_body(x_hbm, sd_hbm, wf_hbm, wb_hbm, part_hbm, zpart_hbm,
               acc, zacc, idx2_0, idx2_1, idx2_2, idx2_3,
               xs0, xs1, xd0, xd1, rows0, rows1,
               wbuf_f, wbuf_r, zloc, zzero, idxrow, wfb, wbb,
               sem_i0, sem_i1, sem_i2, sem_i3,
               sem_s0, sem_s1, sem_d0, sem_d1, sem_c0, sem_c1,
               np_pad, e_pad, ch):
    c = lax.axis_index("c")
    s = lax.axis_index("s")
    wid = c * NS + s
    rows_per_tile = np_pad // NS
    zrows = np_pad // D
    zrows_per_tile = 8
    zero = jnp.zeros((L,), jnp.float32)
    lanes = lax.iota(jnp.int32, L)
    idx2 = [idx2_0, idx2_1, idx2_2, idx2_3]
    xs = [xs0, xs1]
    xd = [xd0, xd1]
    rows = [rows0, rows1]
    sem_i = [sem_i0, sem_i1, sem_i2, sem_i3]
    sem_s = [sem_s0, sem_s1]
    sem_d = [sem_d0, sem_d1]
    sem_c = [sem_c0, sem_c1]

    def zero_row(r, carry):
        for j in range(DV):
            rows0[r, pl.ds(j * L, L)] = zero
        return carry

    lax.fori_loop(0, 2 * K, zero_row, 0)

    def zero_zz(r, carry):
        for j in range(DV):
            zzero[r, pl.ds(j * L, L)] = zero
        return carry

    lax.fori_loop(0, zrows_per_tile, zero_zz, 0)

    tile_base = s * rows_per_tile
    for b in range(rows_per_tile // (2 * K)):
        pltpu.sync_copy(rows0, acc.at[pl.ds(tile_base + b * 2 * K, 2 * K)])

    @pl.when(s < zrows // zrows_per_tile)
    def _():
        pltpu.sync_copy(zzero,
                        zacc.at[pl.ds(s * zrows_per_tile, zrows_per_tile)])

    def zero_z(r, carry):
        for j in range(DV):
            zloc[r, pl.ds(j * L, L)] = zero
        return carry

    lax.fori_loop(0, zrows, zero_z, 0)
    plsc.subcore_barrier()

    pltpu.sync_copy(wf_hbm, wfb)
    pltpu.sync_copy(wb_hbm, wbb)
    wfv = [wfb[pl.ds(j * L, L)] for j in range(DV)]
    wbv = [wbb[pl.ds(j * L, L)] for j in range(DV)]

    epw = ch * K
    base = wid * epw

    def issue_idx(ci, bi):
        off = base + ci * K
        pltpu.async_copy(sd_hbm.at[pl.ds(e_pad + off, K)],
                         idx2[bi].at[pl.ds(0, K)], sem_i[bi])
        pltpu.async_copy(sd_hbm.at[pl.ds(off, K)],
                         idx2[bi].at[pl.ds(K, K)], sem_i[bi])

    def wait_idx(bi):
        pltpu.make_async_copy(sd_hbm.at[pl.ds(0, K)],
                              idx2[bi].at[pl.ds(0, K)], sem_i[bi]).wait()
        pltpu.make_async_copy(sd_hbm.at[pl.ds(0, K)],
                              idx2[bi].at[pl.ds(K, K)], sem_i[bi]).wait()

    def start_gathers(b, bi):
        pltpu.async_copy(x_hbm.at[idx2[bi].at[pl.ds(0, K)]], xd[b], sem_d[b])
        pltpu.async_copy(x_hbm.at[idx2[bi].at[pl.ds(K, K)]], xs[b], sem_s[b])

    def wait_gathers(b, bi):
        pltpu.make_async_copy(x_hbm.at[idx2[bi].at[pl.ds(0, K)]], xd[b],
                              sem_d[b]).wait()
        pltpu.make_async_copy(x_hbm.at[idx2[bi].at[pl.ds(K, K)]], xs[b],
                              sem_s[b]).wait()

    def wait_scatter(b, bi):
        pltpu.make_async_copy(rows[b], acc.at[idx2[bi]], sem_c[b]).wait()

    issue_idx(0, 0)
    wait_idx(0)
    start_gathers(0, 0)
    issue_idx(1, 1)

    def process(ci, b, bi):
        nb = 1 - b
        nbi = (bi + 1) % 4

        @pl.when(ci + 1 < ch)
        def _():
            wait_idx(nbi)
            start_gathers(nb, nbi)

        wait_gathers(b, bi)

        @plsc.parallel_loop(0, K, 1, unroll=1)
        def edge_body(e):
            vs = [xs[b][e, pl.ds(j * L, L)] for j in range(DV)]
            vd = [xd[b][e, pl.ds(j * L, L)] for j in range(DV)]
            accf = vs[0] * vd[0] * wfv[0]
            accb = vs[0] * vd[0] * wbv[0]
            for j in range(1, DV):
                p = vs[j] * vd[j]
                accf = accf + p * wfv[j]
                accb = accb + p * wbv[j]
            wfe = jnp.exp(_leaky(_hsum_bcast(accf)))
            wbe = jnp.exp(_leaky(_hsum_bcast(accb)))
            for j in range(DV):
                rows[b][e, pl.ds(j * L, L)] = vs[j] * wfe
                rows[b][K + e, pl.ds(j * L, L)] = vd[j] * wbe
            wbuf_f[pl.ds(e * L, L)] = wfe
            wbuf_r[pl.ds(e * L, L)] = wbe

        for g in range(K // L):
            ivd = idx2[bi][pl.ds(g * L, L)]
            ivs = idx2[bi][pl.ds(K + g * L, L)]
            iw = lanes * L + (g * L * L)
            wvf = plsc.load_gather(wbuf_f, [iw])
            wvr = plsc.load_gather(wbuf_r, [iw])
            plsc.addupdate_scatter(zloc, [ivd >> 7, ivd & 127], wvf)
            plsc.addupdate_scatter(zloc, [ivs >> 7, ivs & 127], wvr)

        @pl.when(ci >= 1)
        def _():
            wait_scatter(1 - b, (bi + 3) % 4)

        pltpu.async_copy(rows[b], acc.at[idx2[bi]], sem_c[b], add=True)

        @pl.when(ci + 2 < ch)
        def _():
            issue_idx(ci + 2, (bi + 2) % 4)

    def quad_body(t, carry):
        for q in range(4):
            process(4 * t + q, q % 2, q)
        return carry

    nquad = ch // 4
    lax.fori_loop(0, nquad, quad_body, 0)
    for r in range(ch % 4):
        process(nquad * 4 + r, r % 2, r)
    wait_scatter((ch - 1) % 2, (ch - 1) % 4)

    for gg in range(zrows // L):
        idxrow[pl.ds(gg * L, L)] = lanes + gg * L
    pltpu.sync_copy(zloc, zacc.at[idxrow], add=True)

    plsc.subcore_barrier()

    pltpu.sync_copy(acc.at[pl.ds(tile_base, rows_per_tile)],
                    part_hbm.at[pl.ds(c * np_pad + tile_base, rows_per_tile)])

    @pl.when(s < zrows // zrows_per_tile)
    def _():
        pltpu.sync_copy(
            zacc.at[pl.ds(s * zrows_per_tile, zrows_per_tile)],
            zpart_hbm.at[pl.ds(c * zrows + s * zrows_per_tile,
                               zrows_per_tile)])


def _combine_body(part_hbm, zpart_hbm, x_hbm, wf_hbm, mask_hbm, out_hbm,
                  a0, a1, xb, mb, ob, zb0, zb1, zt, wfb, np_pad):
    c = lax.axis_index("c")
    s = lax.axis_index("s")
    wid = c * NS + s
    rows_per_w = np_pad // NW
    base = wid * rows_per_w

    pltpu.sync_copy(wf_hbm, wfb)
    wfv = [wfb[pl.ds(j * L, L)] for j in range(DV)]

    def chunk_body(cb, carry):
        off = base + cb * KB
        pltpu.sync_copy(part_hbm.at[pl.ds(off, KB)], a0)
        pltpu.sync_copy(part_hbm.at[pl.ds(np_pad + off, KB)], a1)
        pltpu.sync_copy(x_hbm.at[pl.ds(off, KB)], xb)
        pltpu.sync_copy(mask_hbm.at[pl.ds(off, KB)], mb.at[pl.ds(0, KB)])
        pltpu.sync_copy(zpart_hbm.at[pl.ds(off, KB)], zb0)
        pltpu.sync_copy(zpart_hbm.at[pl.ds(np_pad + off, KB)], zb1)

        for g in range(KB // L):
            zt[pl.ds(g * L, L)] = (zb0[pl.ds(g * L, L)]
                                   + zb1[pl.ds(g * L, L)])

        def row_body(r, carry2):
            xv = [xb[r, pl.ds(j * L, L)] for j in range(DV)]
            accw = xv[0] * xv[0] * wfv[0]
            for j in range(1, DV):
                accw = accw + xv[j] * xv[j] * wfv[j]
            wsv = jnp.exp(_leaky(_hsum_bcast(accw)))
            z0 = zt[pl.ds(r, L)][0]
            zv = jnp.full((L,), z0, jnp.float32) + wsv
            m0 = mb[pl.ds(r, L)][0]
            pred = jnp.full((L,), m0, jnp.int32) == 1
            for j in range(DV):
                num = a0[r, pl.ds(j * L, L)] + a1[r, pl.ds(j * L, L)] + wsv * xv[j]
                ob[r, pl.ds(j * L, L)] = jnp.where(pred, num / zv + xv[j],
                                                   xv[j] + xv[j])
            return carry2

        lax.fori_loop(0, KB, row_body, 0)
        pltpu.sync_copy(ob, out_hbm.at[pl.ds(off, KB)])
        return carry

    lax.fori_loop(0, rows_per_w // KB, chunk_body, 0)


def kernel(x, edge_index, batch, local_sess_avg, mt_sess_masks,
           W_forward, W_backward, W_alpha):
    n, d = x.shape
    assert d == D
    e = edge_index.shape[1]
    step = NW * KB
    np_pad = ((n + step - 1) // step) * step
    ch = (e + NW * K - 1) // (NW * K)
    e_pad = ch * K * NW
    zrows = np_pad // D

    src = edge_index[0].astype(jnp.int32)
    dst = edge_index[1].astype(jnp.int32)
    pad_idx = jnp.full((e_pad - e,), n, jnp.int32)
    sd = jnp.concatenate([src, pad_idx, dst, pad_idx], axis=0)
    x_pad = jnp.pad(x, ((0, np_pad - n), (0, 0)))
    mask_pad = jnp.pad(mt_sess_masks.astype(jnp.int32), (0, np_pad - n))
    wf = W_forward[:, 0]
    wb = W_backward[:, 0]

    mesh = plsc.VectorSubcoreMesh(core_axis_name="c", subcore_axis_name="s")

    edge_call = pl.kernel(
        functools.partial(_edge_body, np_pad=np_pad, e_pad=e_pad, ch=ch),
        out_type=(jax.ShapeDtypeStruct((NC * np_pad, D), jnp.float32),
                  jax.ShapeDtypeStruct((NC * zrows, D), jnp.float32)),
        mesh=mesh,
        compiler_params=pltpu.CompilerParams(needs_layout_passes=False),
        scratch_types=[
            pltpu.VMEM_SHARED((np_pad, D), jnp.float32),
            pltpu.VMEM_SHARED((zrows, D), jnp.float32),
            pltpu.VMEM((2 * K,), jnp.int32),
            pltpu.VMEM((2 * K,), jnp.int32),
            pltpu.VMEM((2 * K,), jnp.int32),
            pltpu.VMEM((2 * K,), jnp.int32),
            pltpu.VMEM((K, D), jnp.float32),
            pltpu.VMEM((K, D), jnp.float32),
            pltpu.VMEM((K, D), jnp.float32),
            pltpu.VMEM((K, D), jnp.float32),
            pltpu.VMEM((2 * K, D), jnp.float32),
            pltpu.VMEM((2 * K, D), jnp.float32),
            pltpu.VMEM((K * L,), jnp.float32),
            pltpu.VMEM((K * L,), jnp.float32),
            pltpu.VMEM((zrows, D), jnp.float32),
            pltpu.VMEM((8, D), jnp.float32),
            pltpu.VMEM((zrows,), jnp.int32),
            pltpu.VMEM((D,), jnp.float32),
            pltpu.VMEM((D,), jnp.float32),
            pltpu.SemaphoreType.DMA,
            pltpu.SemaphoreType.DMA,
            pltpu.SemaphoreType.DMA,
            pltpu.SemaphoreType.DMA,
            pltpu.SemaphoreType.DMA,
            pltpu.SemaphoreType.DMA,
            pltpu.SemaphoreType.DMA,
            pltpu.SemaphoreType.DMA,
            pltpu.SemaphoreType.DMA,
            pltpu.SemaphoreType.DMA,
        ],
    )
    part, zpart = edge_call(x_pad, sd, wf, wb)
    zflat = zpart.reshape(NC * np_pad)

    combine_call = pl.kernel(
        functools.partial(_combine_body, np_pad=np_pad),
        out_type=jax.ShapeDtypeStruct((np_pad, D), jnp.float32),
        mesh=mesh,
        compiler_params=pltpu.CompilerParams(needs_layout_passes=False),
        scratch_types=[
            pltpu.VMEM((KB, D), jnp.float32),
            pltpu.VMEM((KB, D), jnp.float32),
            pltpu.VMEM((KB, D), jnp.float32),
            pltpu.VMEM((KB + L,), jnp.int32),
            pltpu.VMEM((KB, D), jnp.float32),
            pltpu.VMEM((KB,), jnp.float32),
            pltpu.VMEM((KB,), jnp.float32),
            pltpu.VMEM((KB + L,), jnp.float32),
            pltpu.VMEM((D,), jnp.float32),
        ],
    )
    out_pad = combine_call(part, zflat, x_pad, wf, mask_pad)
    return out_pad[:n]


# speedup vs baseline: 1.5977x; 1.0120x over previous
import functools

import jax
import jax.numpy as jnp
from jax import lax
from jax.experimental import pallas as pl
from jax.experimental.pallas import tpu as pltpu
from jax.experimental.pallas import tpu_sc as plsc

NC = 2
NS = 16
NW = NC * NS
L = 16
D = 128
DV = D // L
K = 32
KB = 160
NEG_SLOPE = 0.01


def _leaky(v):
    return jnp.maximum(v, v * NEG_SLOPE)


def _hsum_bcast(v):
    total = plsc.cumsum(v)[L - 1]
    return jnp.full((L,), total, jnp.float32)


def _edge_body(x_hbm, sd_hbm, wf_hbm, wb_hbm, part_hbm, zpart_hbm,
               acc, zacc, idx2_0, idx2_1, idx2_2, idx2_3,
               xs0, xs1, xd0, xd1, rows0, rows1,
               wbuf_f, wbuf_r, zloc, zzero, idxrow, wfb, wbb,
               sem_i0, sem_i1, sem_i2, sem_i3,
               sem_s0, sem_s1, sem_d0, sem_d1, sem_c0, sem_c1,
               np_pad, e_pad, ch):
    c = lax.axis_index("c")
    s = lax.axis_index("s")
    wid = c * NS + s
    rows_per_tile = np_pad // NS
    zrows = np_pad // D
    zrows_per_tile = 8
    zero = jnp.zeros((L,), jnp.float32)
    lanes = lax.iota(jnp.int32, L)
    idx2 = [idx2_0, idx2_1, idx2_2, idx2_3]
    xs = [xs0, xs1]
    xd = [xd0, xd1]
    rows = [rows0, rows1]
    sem_i = [sem_i0, sem_i1, sem_i2, sem_i3]
    sem_s = [sem_s0, sem_s1]
    sem_d = [sem_d0, sem_d1]
    sem_c = [sem_c0, sem_c1]

    def zero_row(r, carry):
        for j in range(DV):
            rows0[r, pl.ds(j * L, L)] = zero
        return carry

    lax.fori_loop(0, 2 * K, zero_row, 0)

    def zero_zz(r, carry):
        for j in range(DV):
            zzero[r, pl.ds(j * L, L)] = zero
        return carry

    lax.fori_loop(0, zrows_per_tile, zero_zz, 0)

    tile_base = s * rows_per_tile
    for b in range(rows_per_tile // (2 * K)):
        pltpu.sync_copy(rows0, acc.at[pl.ds(tile_base + b * 2 * K, 2 * K)])

    @pl.when(s < zrows // zrows_per_tile)
    def _():
        pltpu.sync_copy(zzero,
                        zacc.at[pl.ds(s * zrows_per_tile, zrows_per_tile)])

    def zero_z(r, carry):
        for j in range(DV):
            zloc[r, pl.ds(j * L, L)] = zero
        return carry

    lax.fori_loop(0, zrows, zero_z, 0)
    plsc.subcore_barrier()

    pltpu.sync_copy(wf_hbm, wfb)
    pltpu.sync_copy(wb_hbm, wbb)
    wfv = [wfb[pl.ds(j * L, L)] for j in range(DV)]
    wbv = [wbb[pl.ds(j * L, L)] for j in range(DV)]

    epw = ch * K
    base = wid * epw

    def issue_idx(ci, bi):
        off = base + ci * K
        pltpu.async_copy(sd_hbm.at[pl.ds(e_pad + off, K)],
                         idx2[bi].at[pl.ds(0, K)], sem_i[bi])
        pltpu.async_copy(sd_hbm.at[pl.ds(off, K)],
                         idx2[bi].at[pl.ds(K, K)], sem_i[bi])

    def wait_idx(bi):
        pltpu.make_async_copy(sd_hbm.at[pl.ds(0, K)],
                              idx2[bi].at[pl.ds(0, K)], sem_i[bi]).wait()
        pltpu.make_async_copy(sd_hbm.at[pl.ds(0, K)],
                              idx2[bi].at[pl.ds(K, K)], sem_i[bi]).wait()

    def start_gathers(b, bi):
        pltpu.async_copy(x_hbm.at[idx2[bi].at[pl.ds(0, K)]], xd[b], sem_d[b])
        pltpu.async_copy(x_hbm.at[idx2[bi].at[pl.ds(K, K)]], xs[b], sem_s[b])

    def wait_gathers(b, bi):
        pltpu.make_async_copy(x_hbm.at[idx2[bi].at[pl.ds(0, K)]], xd[b],
                              sem_d[b]).wait()
        pltpu.make_async_copy(x_hbm.at[idx2[bi].at[pl.ds(K, K)]], xs[b],
                              sem_s[b]).wait()

    def wait_scatter(b, bi):
        pltpu.make_async_copy(rows[b], acc.at[idx2[bi]], sem_c[b]).wait()

    issue_idx(0, 0)
    wait_idx(0)
    start_gathers(0, 0)
    issue_idx(1, 1)

    def process(ci, b, bi):
        nb = 1 - b
        nbi = (bi + 1) % 4

        @pl.when(ci + 1 < ch)
        def _():
            wait_idx(nbi)
            start_gathers(nb, nbi)

        wait_gathers(b, bi)

        @plsc.parallel_loop(0, K, 1, unroll=1)
        def edge_body(e):
            vs = [xs[b][e, pl.ds(j * L, L)] for j in range(DV)]
            vd = [xd[b][e, pl.ds(j * L, L)] for j in range(DV)]
            accf = vs[0] * vd[0] * wfv[0]
            accb = vs[0] * vd[0] * wbv[0]
            for j in range(1, DV):
                p = vs[j] * vd[j]
                accf = accf + p * wfv[j]
                accb = accb + p * wbv[j]
            wfe = jnp.exp(_leaky(_hsum_bcast(accf)))
            wbe = jnp.exp(_leaky(_hsum_bcast(accb)))
            for j in range(DV):
                rows[b][e, pl.ds(j * L, L)] = vs[j] * wfe
                rows[b][K + e, pl.ds(j * L, L)] = vd[j] * wbe
            wbuf_f[pl.ds(e * L, L)] = wfe
            wbuf_r[pl.ds(e * L, L)] = wbe

        for g in range(K // L):
            ivd = idx2[bi][pl.ds(g * L, L)]
            ivs = idx2[bi][pl.ds(K + g * L, L)]
            iw = lanes * L + (g * L * L)
            wvf = plsc.load_gather(wbuf_f, [iw])
            wvr = plsc.load_gather(wbuf_r, [iw])
            plsc.addupdate_scatter(zloc, [ivd >> 7, ivd & 127], wvf)
            plsc.addupdate_scatter(zloc, [ivs >> 7, ivs & 127], wvr)

        @pl.when(ci >= 1)
        def _():
            wait_scatter(1 - b, (bi + 3) % 4)

        pltpu.async_copy(rows[b], acc.at[idx2[bi]], sem_c[b], add=True)

        @pl.when(ci + 2 < ch)
        def _():
            issue_idx(ci + 2, (bi + 2) % 4)

    def quad_body(t, carry):
        for q in range(4):
            process(4 * t + q, q % 2, q)
        return carry

    nquad = ch // 4
    lax.fori_loop(0, nquad, quad_body, 0)
    for r in range(ch % 4):
        process(nquad * 4 + r, r % 2, r)
    wait_scatter((ch - 1) % 2, (ch - 1) % 4)

    for gg in range(zrows // L):
        idxrow[pl.ds(gg * L, L)] = lanes + gg * L
    pltpu.sync_copy(zloc, zacc.at[idxrow], add=True)

    plsc.subcore_barrier()

    pltpu.sync_copy(acc.at[pl.ds(tile_base, rows_per_tile)],
                    part_hbm.at[pl.ds(c * np_pad + tile_base, rows_per_tile)])

    @pl.when(s < zrows // zrows_per_tile)
    def _():
        pltpu.sync_copy(
            zacc.at[pl.ds(s * zrows_per_tile, zrows_per_tile)],
            zpart_hbm.at[pl.ds(c * zrows + s * zrows_per_tile,
                               zrows_per_tile)])


def _combine_body(part_hbm, zpart_hbm, x_hbm, wf_hbm, mask_hbm, out_hbm,
                  a0, a1, xb, mb, ob, zb0, zb1, zt, wfb, np_pad):
    c = lax.axis_index("c")
    s = lax.axis_index("s")
    wid = c * NS + s
    rows_per_w = np_pad // NW
    base = wid * rows_per_w

    pltpu.sync_copy(wf_hbm, wfb)
    wfv = [wfb[pl.ds(j * L, L)] for j in range(DV)]

    def chunk_body(cb, carry):
        off = base + cb * KB
        pltpu.sync_copy(part_hbm.at[pl.ds(off, KB)], a0)
        pltpu.sync_copy(part_hbm.at[pl.ds(np_pad + off, KB)], a1)
        pltpu.sync_copy(x_hbm.at[pl.ds(off, KB)], xb)
        pltpu.sync_copy(mask_hbm.at[pl.ds(off, KB)], mb.at[pl.ds(0, KB)])
        pltpu.sync_copy(zpart_hbm.at[pl.ds(off, KB)], zb0)
        pltpu.sync_copy(zpart_hbm.at[pl.ds(np_pad + off, KB)], zb1)

        for g in range(KB // L):
            zt[pl.ds(g * L, L)] = (zb0[pl.ds(g * L, L)]
                                   + zb1[pl.ds(g * L, L)])

        def row_body(r, carry2):
            xv = [xb[r, pl.ds(j * L, L)] for j in range(DV)]
            accw = xv[0] * xv[0] * wfv[0]
            for j in range(1, DV):
                accw = accw + xv[j] * xv[j] * wfv[j]
            wsv = jnp.exp(_leaky(_hsum_bcast(accw)))
            z0 = zt[pl.ds(r, L)][0]
            zv = jnp.full((L,), z0, jnp.float32) + wsv
            m0 = mb[pl.ds(r, L)][0]
            pred = jnp.full((L,), m0, jnp.int32) == 1
            for j in range(DV):
                num = a0[r, pl.ds(j * L, L)] + a1[r, pl.ds(j * L, L)] + wsv * xv[j]
                ob[r, pl.ds(j * L, L)] = jnp.where(pred, num / zv + xv[j],
                                                   xv[j] + xv[j])
            return carry2

        lax.fori_loop(0, KB, row_body, 0)
        pltpu.sync_copy(ob, out_hbm.at[pl.ds(off, KB)])
        return carry

    lax.fori_loop(0, rows_per_w // KB, chunk_body, 0)


def kernel(x, edge_index, batch, local_sess_avg, mt_sess_masks,
           W_forward, W_backward, W_alpha):
    n, d = x.shape
    assert d == D
    e = edge_index.shape[1]
    step = NW * KB
    np_pad = ((n + step - 1) // step) * step
    ch = (e + NW * K - 1) // (NW * K)
    e_pad = ch * K * NW
    zrows = np_pad // D

    src = edge_index[0].astype(jnp.int32)
    dst = edge_index[1].astype(jnp.int32)
    pad_idx = jnp.full((e_pad - e,), n, jnp.int32)
    sd = jnp.concatenate([src, pad_idx, dst, pad_idx], axis=0)
    x_pad = jnp.pad(x, ((0, np_pad - n), (0, 0)))
    mask_pad = jnp.pad(mt_sess_masks.astype(jnp.int32), (0, np_pad - n))
    wf = W_forward[:, 0]
    wb = W_backward[:, 0]

    mesh = plsc.VectorSubcoreMesh(core_axis_name="c", subcore_axis_name="s")

    edge_call = pl.kernel(
        functools.partial(_edge_body, np_pad=np_pad, e_pad=e_pad, ch=ch),
        out_type=(jax.ShapeDtypeStruct((NC * np_pad, D), jnp.float32),
                  jax.ShapeDtypeStruct((NC * zrows, D), jnp.float32)),
        mesh=mesh,
        compiler_params=pltpu.CompilerParams(needs_layout_passes=False),
        scratch_types=[
            pltpu.VMEM_SHARED((np_pad, D), jnp.float32),
            pltpu.VMEM_SHARED((zrows, D), jnp.float32),
            pltpu.VMEM((2 * K,), jnp.int32),
            pltpu.VMEM((2 * K,), jnp.int32),
            pltpu.VMEM((2 * K,), jnp.int32),
            pltpu.VMEM((2 * K,), jnp.int32),
            pltpu.VMEM((K, D), jnp.float32),
            pltpu.VMEM((K, D), jnp.float32),
            pltpu.VMEM((K, D), jnp.float32),
            pltpu.VMEM((K, D), jnp.float32),
            pltpu.VMEM((2 * K, D), jnp.float32),
            pltpu.VMEM((2 * K, D), jnp.float32),
            pltpu.VMEM((K * L,), jnp.float32),
            pltpu.VMEM((K * L,), jnp.float32),
            pltpu.VMEM((zrows, D), jnp.float32),
            pltpu.VMEM((8, D), jnp.float32),
            pltpu.VMEM((zrows,), jnp.int32),
            pltpu.VMEM((D,), jnp.float32),
            pltpu.VMEM((D,), jnp.float32),
            pltpu.SemaphoreType.DMA,
            pltpu.SemaphoreType.DMA,
            pltpu.SemaphoreType.DMA,
            pltpu.SemaphoreType.DMA,
            pltpu.SemaphoreType.DMA,
            pltpu.SemaphoreType.DMA,
            pltpu.SemaphoreType.DMA,
            pltpu.SemaphoreType.DMA,
            pltpu.SemaphoreType.DMA,
            pltpu.SemaphoreType.DMA,
        ],
    )
    part, zpart = edge_call(x_pad, sd, wf, wb)
    zflat = zpart.reshape(NC * np_pad)

    combine_call = pl.kernel(
        functools.partial(_combine_body, np_pad=np_pad),
        out_type=jax.ShapeDtypeStruct((np_pad, D), jnp.float32),
        mesh=mesh,
        compiler_params=pltpu.CompilerParams(needs_layout_passes=False),
        scratch_types=[
            pltpu.VMEM((KB, D), jnp.float32),
            pltpu.VMEM((KB, D), jnp.float32),
            pltpu.VMEM((KB, D), jnp.float32),
            pltpu.VMEM((KB + L,), jnp.int32),
            pltpu.VMEM((KB, D), jnp.float32),
            pltpu.VMEM((KB,), jnp.float32),
            pltpu.VMEM((KB,), jnp.float32),
            pltpu.VMEM((KB + L,), jnp.float32),
            pltpu.VMEM((D,), jnp.float32),
        ],
    )
    out_pad = combine_call(part, zflat, x_pad, wf, mask_pad)
    return out_pad[:n]
